# Initial kernel scaffold; baseline (speedup 1.0000x reference)
import jax
import jax.numpy as jnp
from jax.experimental import pallas as pl


def kernel(x, edge_index, batch, W0, b0, W1, b1, W2, b2, W3, b3, c1w, c1b, c2w, c2b, m1w, m1b, m2w, m2b):
    raise NotImplementedError("write your pallas kernel here")


# speedup vs baseline: 11.5922x; 11.5922x over previous
import functools

import jax
import jax.numpy as jnp
from jax import lax
from jax.experimental import pallas as pl
from jax.experimental.pallas import tpu as pltpu
from jax.experimental.pallas import tpu_sc as plsc

N = 10000
E = 320000
D = 128
HW = 32
K = 100
B = 64
NP = 10240
TLD = 97
TPAD = 112
ROWT = 1280
NT = NP // ROWT
POOL = K * B
POOL_ACC = 7680
NC_SC = 2
NS_SC = 16
CH = 80


_MESH = dict(core_axis_name="c", subcore_axis_name="s")
_SC_PARAMS = pltpu.CompilerParams(use_tc_tiling_on_sc=False)


def _sc_degree(dst, ones_hbm, z_hbm):
    epc = E // NC_SC
    eps = epc // NS_SC
    nch = eps // CH
    rps = NP // NS_SC

    @functools.partial(
        pl.kernel,
        out_type=jax.ShapeDtypeStruct((NC_SC * NP, 16), jnp.float32),
        mesh=plsc.VectorSubcoreMesh(**_MESH),
        compiler_params=_SC_PARAMS,
        scratch_types=[
            pltpu.VMEM((CH,), jnp.int32),
            pltpu.VMEM((CH, 16), jnp.float32),
            pltpu.VMEM_SHARED((NP, 16), jnp.float32),
            pltpu.SemaphoreType.DMA,
        ],
    )
    def k(dst_hbm, ones_h, z_h, out_hbm, dstv, ones_v, accum, sem):
        cid = lax.axis_index("c")
        sid = lax.axis_index("s")
        pltpu.sync_copy(ones_h, ones_v)
        pltpu.sync_copy(z_h.at[pl.ds(sid * rps, rps)],
                        accum.at[pl.ds(sid * rps, rps)])
        plsc.subcore_barrier()
        base = cid * epc + sid * eps

        @pl.loop(0, nch)
        def _(i):
            pltpu.sync_copy(dst_hbm.at[pl.ds(base + i * CH, CH)], dstv)
            pltpu.sync_copy(ones_v, accum.at[dstv], add=True)

        plsc.subcore_barrier()
        pltpu.sync_copy(accum.at[pl.ds(sid * rps, rps)],
                        out_hbm.at[pl.ds(cid * NP + sid * rps, rps)])

    return k(dst, ones_hbm, z_hbm)


def _sc_edge_scatter(g, src, dst, z_hbm, width):
    epc = E // NC_SC
    eps = epc // NS_SC
    nch = eps // CH
    rps = NP // NS_SC

    @functools.partial(
        pl.kernel,
        out_type=jax.ShapeDtypeStruct((NC_SC * NP, width), jnp.float32),
        mesh=plsc.VectorSubcoreMesh(**_MESH),
        compiler_params=_SC_PARAMS,
        scratch_types=[
            pltpu.VMEM((CH,), jnp.int32),
            pltpu.VMEM((CH,), jnp.int32),
            pltpu.VMEM((CH, width), jnp.float32),
            pltpu.VMEM_SHARED((NP, width), jnp.float32),
            pltpu.SemaphoreType.DMA,
        ],
    )
    def k(g_hbm, src_hbm, dst_hbm, z_h, out_hbm, srcv, dstv, rows, accum, sem):
        cid = lax.axis_index("c")
        sid = lax.axis_index("s")
        pltpu.sync_copy(z_h.at[pl.ds(sid * rps, rps)],
                        accum.at[pl.ds(sid * rps, rps)])
        plsc.subcore_barrier()
        base = cid * epc + sid * eps

        @pl.loop(0, nch)
        def _(i):
            pltpu.sync_copy(src_hbm.at[pl.ds(base + i * CH, CH)], srcv)
            pltpu.sync_copy(dst_hbm.at[pl.ds(base + i * CH, CH)], dstv)
            pltpu.async_copy(g_hbm.at[srcv], rows, sem).wait()
            pltpu.sync_copy(rows, accum.at[dstv], add=True)

        plsc.subcore_barrier()
        pltpu.sync_copy(accum.at[pl.ds(sid * rps, rps)],
                        out_hbm.at[pl.ds(cid * NP + sid * rps, rps)])

    return k(g, src, dst, z_hbm)


def _sc_pool_scatter(hcat, slot, z_hbm):
    rpc = NP // NC_SC
    rpsub = rpc // NS_SC
    nch = rpsub // CH
    zps = POOL_ACC // NS_SC
    ops = POOL // NS_SC

    @functools.partial(
        pl.kernel,
        out_type=jax.ShapeDtypeStruct((NC_SC * POOL, TPAD), jnp.float32),
        mesh=plsc.VectorSubcoreMesh(**_MESH),
        compiler_params=_SC_PARAMS,
        scratch_types=[
            pltpu.VMEM((CH,), jnp.int32),
            pltpu.VMEM((CH, TPAD), jnp.float32),
            pltpu.VMEM_SHARED((POOL_ACC, TPAD), jnp.float32),
            pltpu.SemaphoreType.DMA,
        ],
    )
    def k(h_hbm, slot_hbm, z_h, out_hbm, slotv, rows, accum, sem):
        cid = lax.axis_index("c")
        sid = lax.axis_index("s")
        pltpu.sync_copy(z_h.at[pl.ds(sid * zps, zps)],
                        accum.at[pl.ds(sid * zps, zps)])
        plsc.subcore_barrier()
        base = cid * rpc + sid * rpsub

        @pl.loop(0, nch)
        def _(i):
            pltpu.sync_copy(h_hbm.at[pl.ds(base + i * CH, CH)], rows)
            pltpu.sync_copy(slot_hbm.at[pl.ds(base + i * CH, CH)], slotv)
            pltpu.sync_copy(rows, accum.at[slotv], add=True)

        plsc.subcore_barrier()
        pltpu.sync_copy(accum.at[pl.ds(sid * ops, ops)],
                        out_hbm.at[pl.ds(cid * POOL + sid * ops, ops)])

    return k(hcat, slot, z_hbm)


def _tc_dinv_g0(degp, x, W0):
    def body(degp_ref, x_ref, w_ref, dinv_ref, g0_ref):
        deg = degp_ref[0, :, 0] + degp_ref[1, :, 0] + 1.0
        dinv = lax.rsqrt(jnp.maximum(deg, 1e-12))
        hw = jnp.dot(x_ref[...], w_ref[...], preferred_element_type=jnp.float32)
        dinv_ref[:, 0] = dinv
        g0_ref[...] = dinv[:, None] * hw

    return pl.pallas_call(
        body,
        grid=(NT,),
        in_specs=[
            pl.BlockSpec((2, ROWT, 16), lambda t: (0, t, 0)),
            pl.BlockSpec((ROWT, D), lambda t: (t, 0)),
            pl.BlockSpec((D, HW), lambda t: (0, 0)),
        ],
        out_specs=[
            pl.BlockSpec((ROWT, 1), lambda t: (t, 0)),
            pl.BlockSpec((ROWT, HW), lambda t: (t, 0)),
        ],
        out_shape=[
            jax.ShapeDtypeStruct((NP, 1), jnp.float32),
            jax.ShapeDtypeStruct((NP, HW), jnp.float32),
        ],
    )(degp, x, W0)


def _tc_layer(p, g, dinv, bias, Wn, wout):
    def body(p_ref, g_ref, dinv_ref, b_ref, w_ref, h_ref, gn_ref):
        dinv = dinv_ref[:, 0]
        s = p_ref[0] + p_ref[1] + g_ref[...]
        h = jnp.tanh(dinv[:, None] * s + b_ref[...])
        gn_ref[...] = dinv[:, None] * jnp.dot(
            h, w_ref[...], preferred_element_type=jnp.float32)
        h_ref[...] = h

    return pl.pallas_call(
        body,
        grid=(NT,),
        in_specs=[
            pl.BlockSpec((2, ROWT, HW), lambda t: (0, t, 0)),
            pl.BlockSpec((ROWT, HW), lambda t: (t, 0)),
            pl.BlockSpec((ROWT, 1), lambda t: (t, 0)),
            pl.BlockSpec((1, HW), lambda t: (0, 0)),
            pl.BlockSpec((HW, wout), lambda t: (0, 0)),
        ],
        out_specs=[
            pl.BlockSpec((ROWT, HW), lambda t: (t, 0)),
            pl.BlockSpec((ROWT, wout), lambda t: (t, 0)),
        ],
        out_shape=[
            jax.ShapeDtypeStruct((NP, HW), jnp.float32),
            jax.ShapeDtypeStruct((NP, wout), jnp.float32),
        ],
    )(p, g, dinv, bias, Wn)


def _tc_concat(h1, h2, h3, p3, g3, dinv, b3):
    def body(h1_ref, h2_ref, h3_ref, p3_ref, g3_ref, dinv_ref, b3_ref,
             hcat_ref, key_ref):
        dinv = dinv_ref[:, 0]
        s = p3_ref[0] + p3_ref[1] + g3_ref[...]
        h4 = jnp.tanh(dinv[:, None] * s + b3_ref[0, 0])
        hcat_ref[:, 0:HW] = h1_ref[...]
        hcat_ref[:, HW:2 * HW] = h2_ref[...]
        hcat_ref[:, 2 * HW:3 * HW] = h3_ref[...]
        hcat_ref[:, 3 * HW:TPAD] = jnp.where(
            lax.broadcasted_iota(jnp.int32, (ROWT, 16), 1) == 0, h4, 0.0)
        key_ref[:, 0] = h4[:, 0]

    return pl.pallas_call(
        body,
        grid=(NT,),
        in_specs=[
            pl.BlockSpec((ROWT, HW), lambda t: (t, 0)),
            pl.BlockSpec((ROWT, HW), lambda t: (t, 0)),
            pl.BlockSpec((ROWT, HW), lambda t: (t, 0)),
            pl.BlockSpec((2, ROWT, 16), lambda t: (0, t, 0)),
            pl.BlockSpec((ROWT, 16), lambda t: (t, 0)),
            pl.BlockSpec((ROWT, 1), lambda t: (t, 0)),
            pl.BlockSpec((1, 1), lambda t: (0, 0)),
        ],
        out_specs=[
            pl.BlockSpec((ROWT, TPAD), lambda t: (t, 0)),
            pl.BlockSpec((ROWT, 1), lambda t: (t, 0)),
        ],
        out_shape=[
            jax.ShapeDtypeStruct((NP, TPAD), jnp.float32),
            jax.ShapeDtypeStruct((NP, 1), jnp.float32),
        ],
    )(h1, h2, h3, p3, g3, dinv, b3)


_IT = 256
_JC = 512


def _tc_rank(keyc, batc, keyr, batr):
    def body(kc_ref, bc_ref, kr_ref, br_ref, slot_ref):
        pid = pl.program_id(0)
        i0 = pid * _IT
        ki = kc_ref[...]
        bi = bc_ref[...]
        iidx = i0 + lax.broadcasted_iota(jnp.int32, (_IT, 1), 0)
        ball = br_ref[...]
        bmin = jnp.min(bi)
        bmax = jnp.max(bi)
        jlo = jnp.sum((ball < bmin).astype(jnp.int32))
        jhi = jnp.sum((ball <= bmax).astype(jnp.int32))
        t0 = jlo // _JC
        t1 = (jhi + _JC - 1) // _JC

        def chunk(t, w):
            j0 = t * _JC
            kj = kr_ref[:, pl.ds(j0, _JC)]
            bj = br_ref[:, pl.ds(j0, _JC)]
            jidx = j0 + lax.broadcasted_iota(jnp.int32, (1, _JC), 1)
            eqb = bj == bi
            gt = kj > ki
            tie = (kj == ki) & (jidx < iidx)
            hit = eqb & (gt | tie)
            return w + jnp.sum(hit.astype(jnp.int32), axis=1, keepdims=True)

        w = lax.fori_loop(t0, t1, chunk, jnp.zeros((_IT, 1), jnp.int32))
        valid = (w < K) & (iidx < N)
        dump = POOL + (iidx & 1023)
        slot_ref[...] = jnp.where(valid, bi * K + w, dump)

    return pl.pallas_call(
        body,
        grid=(NP // _IT,),
        in_specs=[
            pl.BlockSpec((_IT, 1), lambda t: (t, 0)),
            pl.BlockSpec((_IT, 1), lambda t: (t, 0)),
            pl.BlockSpec((1, NP), lambda t: (0, 0)),
            pl.BlockSpec((1, NP), lambda t: (0, 0)),
        ],
        out_specs=pl.BlockSpec((_IT, 1), lambda t: (t, 0)),
        out_shape=jax.ShapeDtypeStruct((NP, 1), jnp.int32),
    )(keyc, batc, keyr, batr)


def _tc_head1(pp, c1b2, c1bias, c2d, c2b):
    def body(pp_ref, c1_ref, c1b_ref, c2_ref, c2b_ref, y_ref):
        p = pp_ref[0] + pp_ref[1]
        y1 = jnp.dot(p, c1_ref[...], preferred_element_type=jnp.float32)
        y1 = jnp.maximum(y1 + c1b_ref[...], 0.0)
        y2 = jnp.maximum(y1[:, :16], y1[:, 16:])
        acc = jnp.zeros((POOL // 2, HW), jnp.float32)
        for d in range(5):
            if d == 0:
                sh = y2
            else:
                sh = jnp.concatenate(
                    [y2[d:], jnp.zeros((d, 16), jnp.float32)], axis=0)
            acc = acc + jnp.dot(sh, c2_ref[d],
                                preferred_element_type=jnp.float32)
        y_ref[...] = jnp.maximum(acc + c2b_ref[...], 0.0)

    return pl.pallas_call(
        body,
        in_specs=[
            pl.BlockSpec((2, POOL // 2, 2 * TPAD), lambda: (0, 0, 0)),
            pl.BlockSpec((2 * TPAD, HW), lambda: (0, 0)),
            pl.BlockSpec((1, HW), lambda: (0, 0)),
            pl.BlockSpec((5, 16, HW), lambda: (0, 0, 0)),
            pl.BlockSpec((1, HW), lambda: (0, 0)),
        ],
        out_specs=pl.BlockSpec((POOL // 2, HW), lambda: (0, 0)),
        out_shape=jax.ShapeDtypeStruct((POOL // 2, HW), jnp.float32),
    )(pp, c1b2, c1bias, c2d, c2b)


def _tc_head2(y3f, m1p, m1b, m2w, m2b):
    def body(y_ref, w1_ref, b1_ref, w2_ref, b2_ref, out_ref):
        y4 = jnp.dot(y_ref[...], w1_ref[...],
                     preferred_element_type=jnp.float32)
        y4 = jnp.maximum(y4 + b1_ref[...], 0.0)
        out_ref[...] = jnp.dot(y4, w2_ref[...],
                               preferred_element_type=jnp.float32) + b2_ref[...]

    return pl.pallas_call(
        body,
        in_specs=[
            pl.BlockSpec((B, 1600), lambda: (0, 0)),
            pl.BlockSpec((1600, HW), lambda: (0, 0)),
            pl.BlockSpec((1, HW), lambda: (0, 0)),
            pl.BlockSpec((HW, 2), lambda: (0, 0)),
            pl.BlockSpec((1, 2), lambda: (0, 0)),
        ],
        out_specs=pl.BlockSpec((B, 2), lambda: (0, 0)),
        out_shape=jax.ShapeDtypeStruct((B, 2), jnp.float32),
    )(y3f, m1p, m1b, m2w, m2b)


def kernel(x, edge_index, batch, W0, b0, W1, b1, W2, b2, W3, b3,
           c1w, c1b, c2w, c2b, m1w, m1b, m2w, m2b):
    src = edge_index[0]
    dst = edge_index[1]

    xp = jnp.concatenate([x, jnp.zeros((NP - N, D), jnp.float32)], axis=0)
    bp = jnp.concatenate(
        [batch, jnp.full((NP - N,), 127, jnp.int32)])[:, None]
    ones16 = jnp.ones((CH, 16), jnp.float32)
    zN16 = jnp.zeros((NP, 16), jnp.float32)
    zN32 = jnp.zeros((NP, HW), jnp.float32)
    zPOOL = jnp.zeros((POOL_ACC, TPAD), jnp.float32)
    W3pad = jnp.concatenate([W3, jnp.zeros((HW, 15), jnp.float32)], axis=1)
    c1pad = jnp.concatenate(
        [c1w[:, 0, :], jnp.zeros((16, TPAD - TLD), jnp.float32)], axis=1).T
    c1b2 = jnp.block([[c1pad, jnp.zeros((TPAD, 16), jnp.float32)],
                      [jnp.zeros((TPAD, 16), jnp.float32), c1pad]])
    c1bias2 = jnp.concatenate([c1b, c1b])[None, :]
    c2d = jnp.transpose(c2w, (2, 1, 0))
    m1p = jnp.concatenate(
        [m1w.reshape(32, 46, 32).transpose(1, 0, 2).reshape(46 * 32, 32),
         jnp.zeros((128, 32), jnp.float32)], axis=0)

    degp = _sc_degree(dst, ones16, zN16).reshape(2, NP, 16)
    dinv, g0 = _tc_dinv_g0(degp, xp, W0)

    p0 = _sc_edge_scatter(g0, src, dst, zN32, HW).reshape(2, NP, HW)
    h1, g1 = _tc_layer(p0, g0, dinv, b0[None, :], W1, HW)
    p1 = _sc_edge_scatter(g1, src, dst, zN32, HW).reshape(2, NP, HW)
    h2, g2 = _tc_layer(p1, g1, dinv, b1[None, :], W2, HW)
    p2 = _sc_edge_scatter(g2, src, dst, zN32, HW).reshape(2, NP, HW)
    h3, g3 = _tc_layer(p2, g2, dinv, b2[None, :], W3pad, 16)
    p3 = _sc_edge_scatter(g3, src, dst, zN16, 16).reshape(2, NP, 16)

    hcat, keyv = _tc_concat(h1, h2, h3, p3, g3, dinv, b3[None, :])
    slot = _tc_rank(keyv, bp, keyv.reshape(1, NP), bp.reshape(1, NP))
    pp = _sc_pool_scatter(hcat, slot[:, 0], zPOOL)
    pp = pp.reshape(2, POOL // 2, 2 * TPAD)

    y3 = _tc_head1(pp, c1b2, c1bias2, c2d, c2b[None, :])
    y3f = y3.reshape(B, 1600)
    return _tc_head2(y3f, m1p, m1b[None, :], m2w, m2b[None, :])


# speedup vs baseline: 18.5416x; 1.5995x over previous
import functools

import jax
import jax.numpy as jnp
from jax import lax
from jax.experimental import pallas as pl
from jax.experimental.pallas import tpu as pltpu
from jax.experimental.pallas import tpu_sc as plsc

N = 10000
E = 320000
D = 128
HW = 32
K = 100
B = 64
NP = 10240
TLD = 97
TPAD = 112
ROWT = 1280
NT = NP // ROWT
POOL = K * B
POOL_ACC = 7680
NC_SC = 2
NS_SC = 16
CH = 80


_MESH = dict(core_axis_name="c", subcore_axis_name="s")
_SC_PARAMS = pltpu.CompilerParams(use_tc_tiling_on_sc=False)


def _sc_degree(dst, ones_hbm, z_hbm):
    epc = E // NC_SC
    eps = epc // NS_SC
    nch = eps // CH
    rps = NP // NS_SC

    @functools.partial(
        pl.kernel,
        out_type=jax.ShapeDtypeStruct((NC_SC * NP, 16), jnp.float32),
        mesh=plsc.VectorSubcoreMesh(**_MESH),
        compiler_params=_SC_PARAMS,
        scratch_types=[
            pltpu.VMEM((CH,), jnp.int32),
            pltpu.VMEM((CH, 16), jnp.float32),
            pltpu.VMEM_SHARED((NP, 16), jnp.float32),
            pltpu.SemaphoreType.DMA,
        ],
    )
    def k(dst_hbm, ones_h, z_h, out_hbm, dstv, ones_v, accum, sem):
        cid = lax.axis_index("c")
        sid = lax.axis_index("s")
        pltpu.sync_copy(ones_h, ones_v)
        pltpu.sync_copy(z_h.at[pl.ds(sid * rps, rps)],
                        accum.at[pl.ds(sid * rps, rps)])
        plsc.subcore_barrier()
        base = cid * epc + sid * eps

        @pl.loop(0, nch)
        def _(i):
            pltpu.sync_copy(dst_hbm.at[pl.ds(base + i * CH, CH)], dstv)
            pltpu.sync_copy(ones_v, accum.at[dstv], add=True)

        plsc.subcore_barrier()
        pltpu.sync_copy(accum.at[pl.ds(sid * rps, rps)],
                        out_hbm.at[pl.ds(cid * NP + sid * rps, rps)])

    return k(dst, ones_hbm, z_hbm)


def _sc_edge_scatter(g, src, dst, z_hbm, width):
    epc = E // NC_SC
    eps = epc // NS_SC
    nch = eps // CH
    rps = NP // NS_SC

    @functools.partial(
        pl.kernel,
        out_type=jax.ShapeDtypeStruct((NC_SC * NP, width), jnp.float32),
        mesh=plsc.VectorSubcoreMesh(**_MESH),
        compiler_params=_SC_PARAMS,
        scratch_types=[
            pltpu.VMEM((2, CH), jnp.int32),
            pltpu.VMEM((2, CH), jnp.int32),
            pltpu.VMEM((2, CH, width), jnp.float32),
            pltpu.SemaphoreType.DMA,
            pltpu.SemaphoreType.DMA,
            pltpu.SemaphoreType.DMA,
            pltpu.SemaphoreType.DMA,
            pltpu.VMEM_SHARED((NP, width), jnp.float32),
        ],
    )
    def k(g_hbm, src_hbm, dst_hbm, z_h, out_hbm, srcv, dstv, rows,
          sem_i0, sem_i1, sem_g0, sem_g1, accum):
        cid = lax.axis_index("c")
        sid = lax.axis_index("s")
        pltpu.sync_copy(z_h.at[pl.ds(sid * rps, rps)],
                        accum.at[pl.ds(sid * rps, rps)])
        plsc.subcore_barrier()
        base = cid * epc + sid * eps
        isems = (sem_i0, sem_i1)
        gsems = (sem_g0, sem_g1)

        def idx_start(i, b):
            e0 = base + i * CH
            pltpu.async_copy(src_hbm.at[pl.ds(e0, CH)], srcv.at[b], isems[b])
            pltpu.async_copy(dst_hbm.at[pl.ds(e0, CH)], dstv.at[b], isems[b])

        def idx_wait(i, b):
            e0 = base + i * CH
            pltpu.make_async_copy(src_hbm.at[pl.ds(e0, CH)], srcv.at[b],
                                  isems[b]).wait()
            pltpu.make_async_copy(dst_hbm.at[pl.ds(e0, CH)], dstv.at[b],
                                  isems[b]).wait()

        def gather_start(b):
            pltpu.async_copy(g_hbm.at[srcv.at[b]], rows.at[b], gsems[b])

        def gather_wait(b):
            pltpu.make_async_copy(g_hbm.at[srcv.at[b]], rows.at[b],
                                  gsems[b]).wait()

        def scatter(b):
            pltpu.sync_copy(rows.at[b], accum.at[dstv.at[b]], add=True)

        idx_start(0, 0)

        @pl.loop(0, nch // 2)
        def _(t):
            i0 = 2 * t
            idx_wait(i0, 0)
            gather_start(0)
            idx_start(i0 + 1, 1)
            gather_wait(0)
            idx_wait(i0 + 1, 1)
            gather_start(1)
            scatter(0)
            idx_start(i0 + 2, 0)
            gather_wait(1)
            scatter(1)

        idx_wait(nch - 1, 0)
        gather_start(0)
        gather_wait(0)
        scatter(0)

        plsc.subcore_barrier()
        pltpu.sync_copy(accum.at[pl.ds(sid * rps, rps)],
                        out_hbm.at[pl.ds(cid * NP + sid * rps, rps)])

    return k(g, src, dst, z_hbm)


def _sc_pool_scatter(hcat, slot, z_hbm):
    rpc = NP // NC_SC
    rpsub = rpc // NS_SC
    nch = rpsub // CH
    zps = POOL_ACC // NS_SC
    ops = POOL // NS_SC

    @functools.partial(
        pl.kernel,
        out_type=jax.ShapeDtypeStruct((NC_SC * POOL, TPAD), jnp.float32),
        mesh=plsc.VectorSubcoreMesh(**_MESH),
        compiler_params=_SC_PARAMS,
        scratch_types=[
            pltpu.VMEM((CH,), jnp.int32),
            pltpu.VMEM((CH, TPAD), jnp.float32),
            pltpu.VMEM_SHARED((POOL_ACC, TPAD), jnp.float32),
            pltpu.SemaphoreType.DMA,
        ],
    )
    def k(h_hbm, slot_hbm, z_h, out_hbm, slotv, rows, accum, sem):
        cid = lax.axis_index("c")
        sid = lax.axis_index("s")
        pltpu.sync_copy(z_h.at[pl.ds(sid * zps, zps)],
                        accum.at[pl.ds(sid * zps, zps)])
        plsc.subcore_barrier()
        base = cid * rpc + sid * rpsub

        @pl.loop(0, nch)
        def _(i):
            pltpu.sync_copy(h_hbm.at[pl.ds(base + i * CH, CH)], rows)
            pltpu.sync_copy(slot_hbm.at[pl.ds(base + i * CH, CH)], slotv)
            pltpu.sync_copy(rows, accum.at[slotv], add=True)

        plsc.subcore_barrier()
        pltpu.sync_copy(accum.at[pl.ds(sid * ops, ops)],
                        out_hbm.at[pl.ds(cid * POOL + sid * ops, ops)])

    return k(hcat, slot, z_hbm)


def _tc_dinv_g0(degp, x, W0):
    def body(degp_ref, x_ref, w_ref, dinv_ref, g0_ref):
        deg = degp_ref[0, :, 0] + degp_ref[1, :, 0] + 1.0
        dinv = lax.rsqrt(jnp.maximum(deg, 1e-12))
        hw = jnp.dot(x_ref[...], w_ref[...], preferred_element_type=jnp.float32)
        dinv_ref[:, 0] = dinv
        g0_ref[...] = dinv[:, None] * hw

    return pl.pallas_call(
        body,
        grid=(NT,),
        in_specs=[
            pl.BlockSpec((2, ROWT, 16), lambda t: (0, t, 0)),
            pl.BlockSpec((ROWT, D), lambda t: (t, 0)),
            pl.BlockSpec((D, HW), lambda t: (0, 0)),
        ],
        out_specs=[
            pl.BlockSpec((ROWT, 1), lambda t: (t, 0)),
            pl.BlockSpec((ROWT, HW), lambda t: (t, 0)),
        ],
        out_shape=[
            jax.ShapeDtypeStruct((NP, 1), jnp.float32),
            jax.ShapeDtypeStruct((NP, HW), jnp.float32),
        ],
    )(degp, x, W0)


def _tc_layer(p, g, dinv, bias, Wn, wout):
    def body(p_ref, g_ref, dinv_ref, b_ref, w_ref, h_ref, gn_ref):
        dinv = dinv_ref[:, 0]
        s = p_ref[0] + p_ref[1] + g_ref[...]
        h = jnp.tanh(dinv[:, None] * s + b_ref[...])
        gn_ref[...] = dinv[:, None] * jnp.dot(
            h, w_ref[...], preferred_element_type=jnp.float32)
        h_ref[...] = h

    return pl.pallas_call(
        body,
        grid=(NT,),
        in_specs=[
            pl.BlockSpec((2, ROWT, HW), lambda t: (0, t, 0)),
            pl.BlockSpec((ROWT, HW), lambda t: (t, 0)),
            pl.BlockSpec((ROWT, 1), lambda t: (t, 0)),
            pl.BlockSpec((1, HW), lambda t: (0, 0)),
            pl.BlockSpec((HW, wout), lambda t: (0, 0)),
        ],
        out_specs=[
            pl.BlockSpec((ROWT, HW), lambda t: (t, 0)),
            pl.BlockSpec((ROWT, wout), lambda t: (t, 0)),
        ],
        out_shape=[
            jax.ShapeDtypeStruct((NP, HW), jnp.float32),
            jax.ShapeDtypeStruct((NP, wout), jnp.float32),
        ],
    )(p, g, dinv, bias, Wn)


def _tc_concat(h1, h2, h3, p3, g3, dinv, b3):
    def body(h1_ref, h2_ref, h3_ref, p3_ref, g3_ref, dinv_ref, b3_ref,
             hcat_ref, key_ref):
        dinv = dinv_ref[:, 0]
        s = p3_ref[0] + p3_ref[1] + g3_ref[...]
        h4 = jnp.tanh(dinv[:, None] * s + b3_ref[0, 0])
        hcat_ref[:, 0:HW] = h1_ref[...]
        hcat_ref[:, HW:2 * HW] = h2_ref[...]
        hcat_ref[:, 2 * HW:3 * HW] = h3_ref[...]
        hcat_ref[:, 3 * HW:TPAD] = jnp.where(
            lax.broadcasted_iota(jnp.int32, (ROWT, 16), 1) == 0, h4, 0.0)
        key_ref[:, 0] = h4[:, 0]

    return pl.pallas_call(
        body,
        grid=(NT,),
        in_specs=[
            pl.BlockSpec((ROWT, HW), lambda t: (t, 0)),
            pl.BlockSpec((ROWT, HW), lambda t: (t, 0)),
            pl.BlockSpec((ROWT, HW), lambda t: (t, 0)),
            pl.BlockSpec((2, ROWT, 16), lambda t: (0, t, 0)),
            pl.BlockSpec((ROWT, 16), lambda t: (t, 0)),
            pl.BlockSpec((ROWT, 1), lambda t: (t, 0)),
            pl.BlockSpec((1, 1), lambda t: (0, 0)),
        ],
        out_specs=[
            pl.BlockSpec((ROWT, TPAD), lambda t: (t, 0)),
            pl.BlockSpec((ROWT, 1), lambda t: (t, 0)),
        ],
        out_shape=[
            jax.ShapeDtypeStruct((NP, TPAD), jnp.float32),
            jax.ShapeDtypeStruct((NP, 1), jnp.float32),
        ],
    )(h1, h2, h3, p3, g3, dinv, b3)


_IT = 256
_JC = 512


def _tc_rank(keyc, batc, keyr, batr):
    def body(kc_ref, bc_ref, kr_ref, br_ref, slot_ref):
        pid = pl.program_id(0)
        i0 = pid * _IT
        ki = kc_ref[...]
        bi = bc_ref[...]
        iidx = i0 + lax.broadcasted_iota(jnp.int32, (_IT, 1), 0)
        ball = br_ref[...]
        bmin = jnp.min(bi)
        bmax = jnp.max(bi)
        jlo = jnp.sum((ball < bmin).astype(jnp.int32))
        jhi = jnp.sum((ball <= bmax).astype(jnp.int32))
        t0 = jlo // _JC
        t1 = (jhi + _JC - 1) // _JC

        def chunk(t, w):
            j0 = t * _JC
            kj = kr_ref[:, pl.ds(j0, _JC)]
            bj = br_ref[:, pl.ds(j0, _JC)]
            jidx = j0 + lax.broadcasted_iota(jnp.int32, (1, _JC), 1)
            eqb = bj == bi
            gt = kj > ki
            tie = (kj == ki) & (jidx < iidx)
            hit = eqb & (gt | tie)
            return w + jnp.sum(hit.astype(jnp.int32), axis=1, keepdims=True)

        w = lax.fori_loop(t0, t1, chunk, jnp.zeros((_IT, 1), jnp.int32))
        valid = (w < K) & (iidx < N)
        dump = POOL + (iidx & 1023)
        slot_ref[...] = jnp.where(valid, bi * K + w, dump)

    return pl.pallas_call(
        body,
        grid=(NP // _IT,),
        in_specs=[
            pl.BlockSpec((_IT, 1), lambda t: (t, 0)),
            pl.BlockSpec((_IT, 1), lambda t: (t, 0)),
            pl.BlockSpec((1, NP), lambda t: (0, 0)),
            pl.BlockSpec((1, NP), lambda t: (0, 0)),
        ],
        out_specs=pl.BlockSpec((_IT, 1), lambda t: (t, 0)),
        out_shape=jax.ShapeDtypeStruct((NP, 1), jnp.int32),
    )(keyc, batc, keyr, batr)


def _tc_head1(pp, c1b2, c1bias, c2d, c2b):
    def body(pp_ref, c1_ref, c1b_ref, c2_ref, c2b_ref, y_ref):
        p = pp_ref[0] + pp_ref[1]
        y1 = jnp.dot(p, c1_ref[...], preferred_element_type=jnp.float32)
        y1 = jnp.maximum(y1 + c1b_ref[...], 0.0)
        y2 = jnp.maximum(y1[:, :16], y1[:, 16:])
        acc = jnp.zeros((POOL // 2, HW), jnp.float32)
        for d in range(5):
            if d == 0:
                sh = y2
            else:
                sh = jnp.concatenate(
                    [y2[d:], jnp.zeros((d, 16), jnp.float32)], axis=0)
            acc = acc + jnp.dot(sh, c2_ref[d],
                                preferred_element_type=jnp.float32)
        y_ref[...] = jnp.maximum(acc + c2b_ref[...], 0.0)

    return pl.pallas_call(
        body,
        in_specs=[
            pl.BlockSpec((2, POOL // 2, 2 * TPAD), lambda: (0, 0, 0)),
            pl.BlockSpec((2 * TPAD, HW), lambda: (0, 0)),
            pl.BlockSpec((1, HW), lambda: (0, 0)),
            pl.BlockSpec((5, 16, HW), lambda: (0, 0, 0)),
            pl.BlockSpec((1, HW), lambda: (0, 0)),
        ],
        out_specs=pl.BlockSpec((POOL // 2, HW), lambda: (0, 0)),
        out_shape=jax.ShapeDtypeStruct((POOL // 2, HW), jnp.float32),
    )(pp, c1b2, c1bias, c2d, c2b)


def _tc_head2(y3f, m1p, m1b, m2w, m2b):
    def body(y_ref, w1_ref, b1_ref, w2_ref, b2_ref, out_ref):
        y4 = jnp.dot(y_ref[...], w1_ref[...],
                     preferred_element_type=jnp.float32)
        y4 = jnp.maximum(y4 + b1_ref[...], 0.0)
        out_ref[...] = jnp.dot(y4, w2_ref[...],
                               preferred_element_type=jnp.float32) + b2_ref[...]

    return pl.pallas_call(
        body,
        in_specs=[
            pl.BlockSpec((B, 1600), lambda: (0, 0)),
            pl.BlockSpec((1600, HW), lambda: (0, 0)),
            pl.BlockSpec((1, HW), lambda: (0, 0)),
            pl.BlockSpec((HW, 2), lambda: (0, 0)),
            pl.BlockSpec((1, 2), lambda: (0, 0)),
        ],
        out_specs=pl.BlockSpec((B, 2), lambda: (0, 0)),
        out_shape=jax.ShapeDtypeStruct((B, 2), jnp.float32),
    )(y3f, m1p, m1b, m2w, m2b)


def kernel(x, edge_index, batch, W0, b0, W1, b1, W2, b2, W3, b3,
           c1w, c1b, c2w, c2b, m1w, m1b, m2w, m2b):
    src = edge_index[0]
    dst = edge_index[1]

    xp = jnp.concatenate([x, jnp.zeros((NP - N, D), jnp.float32)], axis=0)
    bp = jnp.concatenate(
        [batch, jnp.full((NP - N,), 127, jnp.int32)])[:, None]
    ones16 = jnp.ones((CH, 16), jnp.float32)
    zN16 = jnp.zeros((NP, 16), jnp.float32)
    zN32 = jnp.zeros((NP, HW), jnp.float32)
    zPOOL = jnp.zeros((POOL_ACC, TPAD), jnp.float32)
    W3pad = jnp.concatenate([W3, jnp.zeros((HW, 15), jnp.float32)], axis=1)
    c1pad = jnp.concatenate(
        [c1w[:, 0, :], jnp.zeros((16, TPAD - TLD), jnp.float32)], axis=1).T
    c1b2 = jnp.block([[c1pad, jnp.zeros((TPAD, 16), jnp.float32)],
                      [jnp.zeros((TPAD, 16), jnp.float32), c1pad]])
    c1bias2 = jnp.concatenate([c1b, c1b])[None, :]
    c2d = jnp.transpose(c2w, (2, 1, 0))
    m1p = jnp.concatenate(
        [m1w.reshape(32, 46, 32).transpose(1, 0, 2).reshape(46 * 32, 32),
         jnp.zeros((128, 32), jnp.float32)], axis=0)

    degp = _sc_degree(dst, ones16, zN16).reshape(2, NP, 16)
    dinv, g0 = _tc_dinv_g0(degp, xp, W0)

    p0 = _sc_edge_scatter(g0, src, dst, zN32, HW).reshape(2, NP, HW)
    h1, g1 = _tc_layer(p0, g0, dinv, b0[None, :], W1, HW)
    p1 = _sc_edge_scatter(g1, src, dst, zN32, HW).reshape(2, NP, HW)
    h2, g2 = _tc_layer(p1, g1, dinv, b1[None, :], W2, HW)
    p2 = _sc_edge_scatter(g2, src, dst, zN32, HW).reshape(2, NP, HW)
    h3, g3 = _tc_layer(p2, g2, dinv, b2[None, :], W3pad, 16)
    p3 = _sc_edge_scatter(g3, src, dst, zN16, 16).reshape(2, NP, 16)

    hcat, keyv = _tc_concat(h1, h2, h3, p3, g3, dinv, b3[None, :])
    slot = _tc_rank(keyv, bp, keyv.reshape(1, NP), bp.reshape(1, NP))
    pp = _sc_pool_scatter(hcat, slot[:, 0], zPOOL)
    pp = pp.reshape(2, POOL // 2, 2 * TPAD)

    y3 = _tc_head1(pp, c1b2, c1bias2, c2d, c2b[None, :])
    y3f = y3.reshape(B, 1600)
    return _tc_head2(y3f, m1p, m1b[None, :], m2w, m2b[None, :])
